# baseline (speedup 1.0000x reference)
import functools

import jax
import jax.numpy as jnp
from jax import lax
from jax.experimental import pallas as pl
from jax.experimental.pallas import tpu as pltpu
from jax.experimental.pallas import tpu_sc as plsc

V = 100000
D = 768
E = 64
K = 1
DFF = 4 * D
B = 1
S = 2048
T = B * S
CAP = 2 * K * B * S // E
SLOTS = E * CAP

NC = 2
NS = 16
NW = NC * NS
L = 16


def _sc_mesh():
    return plsc.VectorSubcoreMesh(core_axis_name="c", subcore_axis_name="s")


_SC_PARAMS = pltpu.CompilerParams(needs_layout_passes=False)


def _sc_wid():
    return lax.axis_index("s") * NC + lax.axis_index("c")


def _sc_gather(table, idx, n_rows, d):
    rpw = n_rows // NW

    @functools.partial(
        pl.kernel,
        out_type=jax.ShapeDtypeStruct((n_rows, d), jnp.float32),
        mesh=_sc_mesh(),
        compiler_params=_SC_PARAMS,
        scratch_types=[
            pltpu.VMEM((rpw,), jnp.int32),
            pltpu.VMEM((rpw, d), jnp.float32),
            pltpu.SemaphoreType.DMA,
        ],
    )
    def k(table_hbm, idx_hbm, out_hbm, idx_v, rows_v, sem):
        base = _sc_wid() * rpw
        pltpu.sync_copy(idx_hbm.at[pl.ds(base, rpw)], idx_v)
        pltpu.async_copy(table_hbm.at[idx_v], rows_v, sem).wait()
        pltpu.sync_copy(rows_v, out_hbm.at[pl.ds(base, rpw)])

    return k(table, idx)


_TBLK = 128


def _router(h, Wr, br):
    nb = T // _TBLK

    def body(h_ref, wr_ref, br_ref, t4s_ref, cslot_ref, gk_ref, cnt_ref, acc_ref):
        b = pl.program_id(0)

        @pl.when(b == 0)
        def _():
            cnt_ref[...] = jnp.zeros_like(cnt_ref)
            acc_ref[...] = jnp.zeros_like(acc_ref)

        logits = (
            jnp.dot(h_ref[...], wr_ref[...], preferred_element_type=jnp.float32)
            + br_ref[...]
        )
        m = jnp.max(logits, axis=1, keepdims=True)
        denom = jnp.sum(jnp.exp(logits - m), axis=1)
        topv = 1.0 / denom
        ids = lax.broadcasted_iota(jnp.int32, (_TBLK, E), 1)
        eidx = jnp.min(jnp.where(logits == m, ids, E), axis=1)
        oh = (ids == eidx[:, None]).astype(jnp.float32)
        ti = lax.broadcasted_iota(jnp.int32, (_TBLK, _TBLK), 0)
        tj = lax.broadcasted_iota(jnp.int32, (_TBLK, _TBLK), 1)
        lt = (tj < ti).astype(jnp.float32)
        pos = cnt_ref[...] + jnp.dot(lt, oh, preferred_element_type=jnp.float32)
        cnt_ref[...] = cnt_ref[...] + jnp.sum(oh, axis=0, keepdims=True)
        posn = jnp.sum(pos * oh, axis=1)
        keep = posn < CAP
        s = eidx * CAP + posn.astype(jnp.int32)
        cslot_ref[...] = jnp.where(keep, s, 0)
        gk_ref[...] = (topv * keep.astype(jnp.float32))[:, None]
        pid = lax.broadcasted_iota(jnp.int32, (_TBLK, CAP), 1)
        ohp = (pid == posn.astype(jnp.int32)[:, None]).astype(jnp.float32)
        tokp1 = (
            lax.broadcasted_iota(jnp.int32, (_TBLK, 1), 0) + (b * _TBLK + 1)
        ).astype(jnp.float32)
        acc_ref[...] += lax.dot_general(
            oh * tokp1, ohp, (((0,), (0,)), ((), ())),
            preferred_element_type=jnp.float32,
        )

        @pl.when(b == nb - 1)
        def _():
            se = lax.broadcasted_iota(jnp.int32, (E, CAP), 0)
            sp = lax.broadcasted_iota(jnp.int32, (E, CAP), 1)
            t4s_ref[...] = jnp.where(
                acc_ref[...] > 0.0,
                acc_ref[...].astype(jnp.int32) - 1,
                (se * CAP + sp) & (T - 1),
            )

    return pl.pallas_call(
        body,
        grid=(nb,),
        in_specs=[
            pl.BlockSpec((_TBLK, D), lambda b: (b, 0)),
            pl.BlockSpec((D, E), lambda b: (0, 0)),
            pl.BlockSpec((1, E), lambda b: (0, 0)),
        ],
        out_specs=[
            pl.BlockSpec((E, CAP), lambda b: (0, 0)),
            pl.BlockSpec((_TBLK,), lambda b: (b,)),
            pl.BlockSpec((_TBLK, 1), lambda b: (b, 0)),
        ],
        out_shape=[
            jax.ShapeDtypeStruct((E, CAP), jnp.int32),
            jax.ShapeDtypeStruct((T,), jnp.int32),
            jax.ShapeDtypeStruct((T, 1), jnp.float32),
        ],
        scratch_shapes=[
            pltpu.VMEM((1, E), jnp.float32),
            pltpu.VMEM((E, CAP), jnp.float32),
        ],
    )(h, Wr, br.reshape(1, E))


_FBLK = 3072


def _ffn(X, W1, b1, W2, b2):
    nf = DFF // _FBLK

    def body(x_ref, w1_ref, b1_ref, w2_ref, b2_ref, y_ref):
        f = pl.program_id(1)
        hid = jnp.maximum(
            jnp.dot(x_ref[0], w1_ref[0], preferred_element_type=jnp.float32)
            + b1_ref[0, 0][None, :],
            0.0,
        )
        part = jnp.dot(hid, w2_ref[0], preferred_element_type=jnp.float32)

        @pl.when(f == 0)
        def _():
            y_ref[0] = part + b2_ref[0, 0][None, :]

        @pl.when(f != 0)
        def _():
            y_ref[0] += part

    return pl.pallas_call(
        body,
        grid=(E, nf),
        in_specs=[
            pl.BlockSpec((1, CAP, D), lambda e, f: (e, 0, 0)),
            pl.BlockSpec((1, D, _FBLK), lambda e, f: (e, 0, f)),
            pl.BlockSpec((1, 1, _FBLK), lambda e, f: (e, 0, f)),
            pl.BlockSpec((1, _FBLK, D), lambda e, f: (e, f, 0)),
            pl.BlockSpec((1, 1, D), lambda e, f: (e, 0, 0)),
        ],
        out_specs=pl.BlockSpec((1, CAP, D), lambda e, f: (e, 0, 0)),
        out_shape=jax.ShapeDtypeStruct((E, CAP, D), jnp.float32),
    )(X.reshape(E, CAP, D), W1, b1.reshape(E, 1, DFF), W2, b2.reshape(E, 1, D))


_VBLK = 2048


def _proj_t(xsT, WoT, bor):
    nv = pl.cdiv(V, _VBLK)

    def body(xs_ref, wo_ref, bo_ref, out_ref):
        boc = jnp.swapaxes(bo_ref[...], 0, 1)
        out_ref[...] = (
            jnp.dot(
                wo_ref[...].astype(jnp.bfloat16),
                xs_ref[...],
                preferred_element_type=jnp.float32,
            )
            + boc
        )

    return pl.pallas_call(
        body,
        grid=(nv,),
        in_specs=[
            pl.BlockSpec((D, T), lambda v: (0, 0)),
            pl.BlockSpec((_VBLK, D), lambda v: (v, 0)),
            pl.BlockSpec((1, _VBLK), lambda v: (0, v)),
        ],
        out_specs=pl.BlockSpec((_VBLK, T), lambda v: (v, 0)),
        out_shape=jax.ShapeDtypeStruct((V, T), jnp.float32),
    )(xsT, WoT, bor)


def kernel(x, emb, Wr, br, W1, b1, W2, b2, Wo, bo):
    xf = x.reshape(T)
    h = _sc_gather(emb, xf, T, D)
    t4s, cslot, gk = _router(h, Wr, br)
    X = _sc_gather(h, t4s.reshape(SLOTS), SLOTS, D)
    Y = _ffn(X, W1, b1, W2, b2)
    moe = _sc_gather(Y.reshape(SLOTS, D), cslot, T, D)
    xsT = jnp.swapaxes((moe * gk).astype(jnp.bfloat16), 0, 1)
    y_t = _proj_t(xsT, jnp.swapaxes(Wo, 0, 1), bo.reshape(1, V))
    return jnp.swapaxes(y_t, 0, 1).reshape(B, S, V)


# speedup vs baseline: 1.0055x; 1.0055x over previous
import functools

import jax
import jax.numpy as jnp
from jax import lax
from jax.experimental import pallas as pl
from jax.experimental.pallas import tpu as pltpu
from jax.experimental.pallas import tpu_sc as plsc

V = 100000
D = 768
E = 64
K = 1
DFF = 4 * D
B = 1
S = 2048
T = B * S
CAP = 2 * K * B * S // E
SLOTS = E * CAP

NC = 2
NS = 16
NW = NC * NS
L = 16


def _sc_mesh():
    return plsc.VectorSubcoreMesh(core_axis_name="c", subcore_axis_name="s")


_SC_PARAMS = pltpu.CompilerParams(needs_layout_passes=False)


def _sc_wid():
    return lax.axis_index("s") * NC + lax.axis_index("c")


def _sc_gather(table, idx, n_rows, d):
    rpw = n_rows // NW

    @functools.partial(
        pl.kernel,
        out_type=jax.ShapeDtypeStruct((n_rows, d), jnp.float32),
        mesh=_sc_mesh(),
        compiler_params=_SC_PARAMS,
        scratch_types=[
            pltpu.VMEM((rpw,), jnp.int32),
            pltpu.VMEM((rpw, d), jnp.float32),
            pltpu.SemaphoreType.DMA,
        ],
    )
    def k(table_hbm, idx_hbm, out_hbm, idx_v, rows_v, sem):
        base = _sc_wid() * rpw
        pltpu.sync_copy(idx_hbm.at[pl.ds(base, rpw)], idx_v)
        pltpu.async_copy(table_hbm.at[idx_v], rows_v, sem).wait()
        pltpu.sync_copy(rows_v, out_hbm.at[pl.ds(base, rpw)])

    return k(table, idx)


_TBLK = 256


def _router(h, Wr, br):
    nb = T // _TBLK

    def body(h_ref, wr_ref, br_ref, t4s_ref, cslot_ref, gk_ref, cnt_ref, acc_ref):
        b = pl.program_id(0)

        @pl.when(b == 0)
        def _():
            cnt_ref[...] = jnp.zeros_like(cnt_ref)
            acc_ref[...] = jnp.zeros_like(acc_ref)

        logits = (
            jnp.dot(h_ref[...], wr_ref[...], preferred_element_type=jnp.float32)
            + br_ref[...]
        )
        m = jnp.max(logits, axis=1, keepdims=True)
        denom = jnp.sum(jnp.exp(logits - m), axis=1)
        topv = 1.0 / denom
        ids = lax.broadcasted_iota(jnp.int32, (_TBLK, E), 1)
        eidx = jnp.min(jnp.where(logits == m, ids, E), axis=1)
        oh = (ids == eidx[:, None]).astype(jnp.float32)
        ti = lax.broadcasted_iota(jnp.int32, (_TBLK, _TBLK), 0)
        tj = lax.broadcasted_iota(jnp.int32, (_TBLK, _TBLK), 1)
        lt = (tj < ti).astype(jnp.float32)
        pos = cnt_ref[...] + jnp.dot(lt, oh, preferred_element_type=jnp.float32)
        cnt_ref[...] = cnt_ref[...] + jnp.sum(oh, axis=0, keepdims=True)
        posn = jnp.sum(pos * oh, axis=1)
        keep = posn < CAP
        s = eidx * CAP + posn.astype(jnp.int32)
        cslot_ref[...] = jnp.where(keep, s, 0)
        gk_ref[...] = (topv * keep.astype(jnp.float32))[:, None]
        s_eff = jnp.where(keep, s, SLOTS)[:, None]
        sid = lax.broadcasted_iota(jnp.int32, (_TBLK, SLOTS), 1)
        tokp1 = (
            lax.broadcasted_iota(jnp.int32, (_TBLK, 1), 0) + (b * _TBLK + 1)
        ).astype(jnp.float32)
        acc_ref[...] += jnp.sum(
            jnp.where(sid == s_eff, tokp1, 0.0), axis=0, keepdims=True
        )

        @pl.when(b == nb - 1)
        def _():
            sall = lax.broadcasted_iota(jnp.int32, (1, SLOTS), 1)
            t4s_ref[...] = jnp.where(
                acc_ref[...] > 0.0,
                acc_ref[...].astype(jnp.int32) - 1,
                sall & (T - 1),
            )

    return pl.pallas_call(
        body,
        grid=(nb,),
        in_specs=[
            pl.BlockSpec((_TBLK, D), lambda b: (b, 0)),
            pl.BlockSpec((D, E), lambda b: (0, 0)),
            pl.BlockSpec((1, E), lambda b: (0, 0)),
        ],
        out_specs=[
            pl.BlockSpec((1, SLOTS), lambda b: (0, 0)),
            pl.BlockSpec((_TBLK,), lambda b: (b,)),
            pl.BlockSpec((_TBLK, 1), lambda b: (b, 0)),
        ],
        out_shape=[
            jax.ShapeDtypeStruct((1, SLOTS), jnp.int32),
            jax.ShapeDtypeStruct((T,), jnp.int32),
            jax.ShapeDtypeStruct((T, 1), jnp.float32),
        ],
        scratch_shapes=[
            pltpu.VMEM((1, E), jnp.float32),
            pltpu.VMEM((1, SLOTS), jnp.float32),
        ],
    )(h, Wr, br.reshape(1, E))


_FBLK = 3072


def _ffn(X, W1, b1, W2, b2):
    nf = DFF // _FBLK

    def body(x_ref, w1_ref, b1_ref, w2_ref, b2_ref, y_ref):
        f = pl.program_id(1)
        hid = jnp.maximum(
            jnp.dot(x_ref[0], w1_ref[0], preferred_element_type=jnp.float32)
            + b1_ref[0, 0][None, :],
            0.0,
        )
        part = jnp.dot(hid, w2_ref[0], preferred_element_type=jnp.float32)

        @pl.when(f == 0)
        def _():
            y_ref[0] = part + b2_ref[0, 0][None, :]

        @pl.when(f != 0)
        def _():
            y_ref[0] += part

    return pl.pallas_call(
        body,
        grid=(E, nf),
        in_specs=[
            pl.BlockSpec((1, CAP, D), lambda e, f: (e, 0, 0)),
            pl.BlockSpec((1, D, _FBLK), lambda e, f: (e, 0, f)),
            pl.BlockSpec((1, 1, _FBLK), lambda e, f: (e, 0, f)),
            pl.BlockSpec((1, _FBLK, D), lambda e, f: (e, f, 0)),
            pl.BlockSpec((1, 1, D), lambda e, f: (e, 0, 0)),
        ],
        out_specs=pl.BlockSpec((1, CAP, D), lambda e, f: (e, 0, 0)),
        out_shape=jax.ShapeDtypeStruct((E, CAP, D), jnp.float32),
    )(X.reshape(E, CAP, D), W1, b1.reshape(E, 1, DFF), W2, b2.reshape(E, 1, D))


_VBLK = 2048


def _proj_t(xsT, WoT, bor):
    nv = pl.cdiv(V, _VBLK)

    def body(xs_ref, wo_ref, bo_ref, out_ref):
        boc = jnp.swapaxes(bo_ref[...], 0, 1)
        out_ref[...] = (
            jnp.dot(
                wo_ref[...].astype(jnp.bfloat16),
                xs_ref[...],
                preferred_element_type=jnp.float32,
            )
            + boc
        )

    return pl.pallas_call(
        body,
        grid=(nv,),
        in_specs=[
            pl.BlockSpec((D, T), lambda v: (0, 0)),
            pl.BlockSpec((_VBLK, D), lambda v: (v, 0)),
            pl.BlockSpec((1, _VBLK), lambda v: (0, v)),
        ],
        out_specs=pl.BlockSpec((_VBLK, T), lambda v: (v, 0)),
        out_shape=jax.ShapeDtypeStruct((V, T), jnp.float32),
    )(xsT, WoT, bor)


def kernel(x, emb, Wr, br, W1, b1, W2, b2, Wo, bo):
    xf = x.reshape(T)
    h = _sc_gather(emb, xf, T, D)
    t4s, cslot, gk = _router(h, Wr, br)
    X = _sc_gather(h, t4s.reshape(SLOTS), SLOTS, D)
    Y = _ffn(X, W1, b1, W2, b2)
    moe = _sc_gather(Y.reshape(SLOTS, D), cslot, T, D)
    xsT = jnp.swapaxes((moe * gk).astype(jnp.bfloat16), 0, 1)
    y_t = _proj_t(xsT, jnp.swapaxes(Wo, 0, 1), bo.reshape(1, V))
    return jnp.swapaxes(y_t, 0, 1).reshape(B, S, V)
